# Initial kernel scaffold; baseline (speedup 1.0000x reference)
import jax
import jax.numpy as jnp
from jax.experimental import pallas as pl


def kernel(x, edge_index, W0, b0, W1, b1, W2, b2, Wr, br):
    raise NotImplementedError("write your pallas kernel here")


# speedup vs baseline: 2.7509x; 2.7509x over previous
import functools

import jax
import jax.numpy as jnp
from jax import lax
from jax.experimental import pallas as pl
from jax.experimental.pallas import tpu as pltpu
from jax.experimental.pallas import tpu_sc as plsc

N_NODES = 10000
D = 128
N_PAD = 10240
E = 320000
E_PAD = 327680
NC = 2
NS = 16
CHUNK = 128
EPC = E_PAD // NC
EPS = EPC // NS
NCHUNKS = EPS // CHUNK
RPS = N_PAD // NS
DW = 16
BLK = 1024
GRID = N_PAD // BLK


def _sc_mesh():
    return plsc.VectorSubcoreMesh(core_axis_name="c", subcore_axis_name="s",
                                  num_cores=NC, num_subcores=NS)


N_DEG = 2 * N_PAD
DEG_RPS = N_DEG // NS
DEG_EPS = 2 * EPS
DEG_NCHUNKS = DEG_EPS // CHUNK


@functools.cache
def _get_deg_kernel():
    return functools.partial(
        pl.kernel,
        out_type=jax.ShapeDtypeStruct((NC * N_DEG,), jnp.float32),
        mesh=_sc_mesh(),
        scratch_types=[
            pltpu.VMEM_SHARED((N_DEG,), jnp.float32),
            pltpu.VMEM((CHUNK,), jnp.int32),
            pltpu.VMEM((CHUNK,), jnp.float32),
            pltpu.VMEM((DEG_RPS,), jnp.float32),
        ],
    )(_deg_body)


def _deg_body(sd_idx, ones_h, zeros_h, deg_out,
              deg_s, idx_v, ones_v, buf):
    c = lax.axis_index("c")
    s = lax.axis_index("s")
    r0 = s * DEG_RPS
    pltpu.sync_copy(ones_h, ones_v)
    pltpu.sync_copy(zeros_h, buf)
    pltpu.sync_copy(buf, deg_s.at[pl.ds(r0, DEG_RPS)])
    plsc.subcore_barrier()

    def body(t, carry):
        base = c * (2 * EPC) + s * DEG_EPS + t * CHUNK
        pltpu.sync_copy(sd_idx.at[pl.ds(base, CHUNK)], idx_v)
        pltpu.sync_copy(ones_v, deg_s.at[idx_v], add=True)
        return carry

    lax.fori_loop(0, DEG_NCHUNKS, body, 0)
    plsc.subcore_barrier()
    pltpu.sync_copy(deg_s.at[pl.ds(r0, DEG_RPS)], buf)
    pltpu.sync_copy(buf, deg_out.at[pl.ds(c * N_DEG + r0, DEG_RPS)])


@functools.cache
def _get_edge_kernel():
    return functools.partial(
        pl.kernel,
        out_type=jax.ShapeDtypeStruct((NC * N_PAD, D), jnp.float32),
        mesh=_sc_mesh(),
        scratch_types=[
            pltpu.VMEM_SHARED((N_PAD, D), jnp.float32),
            pltpu.VMEM((CHUNK,), jnp.int32),
            pltpu.VMEM((CHUNK,), jnp.int32),
            pltpu.VMEM((CHUNK, D), jnp.float32),
        ],
    )(_edge_body)


def _edge_body(hpre, srcp, dstp, zeros_h, acc_out,
               acc_s, idx_s, idx_d, rows):
    c = lax.axis_index("c")
    s = lax.axis_index("s")
    r0 = s * RPS
    pltpu.sync_copy(zeros_h, rows)
    for k in range(RPS // CHUNK):
        pltpu.sync_copy(rows, acc_s.at[pl.ds(r0 + k * CHUNK, CHUNK)])
    plsc.subcore_barrier()

    def body(t, carry):
        base = c * EPC + s * EPS + t * CHUNK
        pltpu.sync_copy(srcp.at[pl.ds(base, CHUNK)], idx_s)
        pltpu.sync_copy(dstp.at[pl.ds(base, CHUNK)], idx_d)
        pltpu.sync_copy(hpre.at[idx_s], rows)
        pltpu.sync_copy(rows, acc_s.at[idx_d], add=True)
        return carry

    lax.fori_loop(0, NCHUNKS, body, 0)
    plsc.subcore_barrier()
    for k in range(RPS // CHUNK):
        pltpu.sync_copy(acc_s.at[pl.ds(r0 + k * CHUNK, CHUNK)], rows)
        pltpu.sync_copy(rows, acc_out.at[pl.ds(c * N_PAD + r0 + k * CHUNK, CHUNK)])


def _prep_body(x_ref, dego, degi, hpre_ref, sin_ref, sout_ref):
    so = lax.rsqrt(jnp.maximum(dego[...], 1.0))
    si = lax.rsqrt(jnp.maximum(degi[...], 1.0))
    hpre_ref[...] = x_ref[...] * so
    sout_ref[...] = so
    sin_ref[...] = si


def _tc_prep(x_pad, deg_all):
    return pl.pallas_call(
        _prep_body,
        grid=(GRID,),
        in_specs=[
            pl.BlockSpec((BLK, D), lambda i: (i, 0)),
            pl.BlockSpec((BLK, 1), lambda i: (i, 0)),
            pl.BlockSpec((BLK, 1), lambda i: (i + 3 * GRID, 0)),
        ],
        out_specs=[
            pl.BlockSpec((BLK, D), lambda i: (i, 0)),
            pl.BlockSpec((BLK, 1), lambda i: (i, 0)),
            pl.BlockSpec((BLK, 1), lambda i: (i, 0)),
        ],
        out_shape=[
            jax.ShapeDtypeStruct((N_PAD, D), jnp.float32),
            jax.ShapeDtypeStruct((N_PAD, 1), jnp.float32),
            jax.ShapeDtypeStruct((N_PAD, 1), jnp.float32),
        ],
    )(x_pad, deg_all, deg_all)


def _layer_body(p0, p1, sin_ref, sout_ref, w_ref, b_ref, out_ref):
    agg = (p0[...] + p1[...]) * sin_ref[...]
    h = jnp.dot(agg, w_ref[...], preferred_element_type=jnp.float32) + b_ref[...]
    out_ref[...] = jnp.maximum(h, 0.0) * sout_ref[...]


def _tc_layer(acc_partials, sin, sout, w, b):
    return pl.pallas_call(
        _layer_body,
        grid=(GRID,),
        in_specs=[
            pl.BlockSpec((BLK, D), lambda i: (i, 0)),
            pl.BlockSpec((BLK, D), lambda i: (i + GRID, 0)),
            pl.BlockSpec((BLK, 1), lambda i: (i, 0)),
            pl.BlockSpec((BLK, 1), lambda i: (i, 0)),
            pl.BlockSpec((D, D), lambda i: (0, 0)),
            pl.BlockSpec((1, D), lambda i: (0, 0)),
        ],
        out_specs=pl.BlockSpec((BLK, D), lambda i: (i, 0)),
        out_shape=jax.ShapeDtypeStruct((N_PAD, D), jnp.float32),
    )(acc_partials, acc_partials, sin, sout, w, b.reshape(1, D))


def _head_body(p0, p1, sin_ref, w_ref, b_ref, wr_ref, br_ref, out_ref, acc_ref):
    i = pl.program_id(0)
    agg = (p0[...] + p1[...]) * sin_ref[...]
    h = jnp.dot(agg, w_ref[...], preferred_element_type=jnp.float32) + b_ref[...]
    h = jnp.maximum(h, 0.0)
    row = i * BLK + lax.broadcasted_iota(jnp.int32, (BLK, 1), 0)
    h = jnp.where(row < N_NODES, h, 0.0)
    colsum = jnp.sum(h, axis=0, keepdims=True)

    @pl.when(i == 0)
    def _():
        acc_ref[...] = colsum

    @pl.when(i > 0)
    def _():
        acc_ref[...] = acc_ref[...] + colsum

    @pl.when(i == GRID - 1)
    def _():
        hg = acc_ref[...] * (1.0 / N_NODES)
        out_ref[...] = (jnp.dot(hg, wr_ref[...],
                                preferred_element_type=jnp.float32)
                        + br_ref[...])


def _tc_head(acc_partials, sin, w, b, wr, br):
    return pl.pallas_call(
        _head_body,
        grid=(GRID,),
        in_specs=[
            pl.BlockSpec((BLK, D), lambda i: (i, 0)),
            pl.BlockSpec((BLK, D), lambda i: (i + GRID, 0)),
            pl.BlockSpec((BLK, 1), lambda i: (i, 0)),
            pl.BlockSpec((D, D), lambda i: (0, 0)),
            pl.BlockSpec((1, D), lambda i: (0, 0)),
            pl.BlockSpec((D, 1), lambda i: (0, 0)),
            pl.BlockSpec((1, 1), lambda i: (0, 0)),
        ],
        out_specs=pl.BlockSpec((1, 1), lambda i: (0, 0)),
        out_shape=jax.ShapeDtypeStruct((1, 1), jnp.float32),
        scratch_shapes=[pltpu.VMEM((1, D), jnp.float32)],
    )(acc_partials, acc_partials, sin, w, b.reshape(1, D), wr,
      br.reshape(1, 1))


def kernel(x, edge_index, W0, b0, W1, b1, W2, b2, Wr, br):
    src = edge_index[0].astype(jnp.int32)
    dst = edge_index[1].astype(jnp.int32)
    pad_idx = jnp.full((E_PAD - E,), N_NODES, dtype=jnp.int32)
    srcp = jnp.concatenate([src, pad_idx])
    dstp = jnp.concatenate([dst, pad_idx])
    x_pad = jnp.pad(x, ((0, N_PAD - N_NODES), (0, 0)))

    ones_h = jnp.ones((CHUNK,), jnp.float32)
    zeros_deg = jnp.zeros((DEG_RPS,), jnp.float32)
    zeros_row = jnp.zeros((CHUNK, D), jnp.float32)
    sd_idx = jnp.concatenate([srcp, dstp + N_PAD])

    deg_kernel = _get_deg_kernel()
    edge_kernel = _get_edge_kernel()
    deg_all = deg_kernel(sd_idx, ones_h, zeros_deg)
    hpre, sin, sout = _tc_prep(x_pad, deg_all.reshape(NC * N_DEG, 1))

    acc1 = edge_kernel(hpre, srcp, dstp, zeros_row)
    hpre2 = _tc_layer(acc1, sin, sout, W0, b0)
    acc2 = edge_kernel(hpre2, srcp, dstp, zeros_row)
    hpre3 = _tc_layer(acc2, sin, sout, W1, b1)
    acc3 = edge_kernel(hpre3, srcp, dstp, zeros_row)
    return _tc_head(acc3, sin, W2, b2, Wr, br)


# speedup vs baseline: 3.2793x; 1.1921x over previous
import functools

import jax
import jax.numpy as jnp
from jax import lax
from jax.experimental import pallas as pl
from jax.experimental.pallas import tpu as pltpu
from jax.experimental.pallas import tpu_sc as plsc

N_NODES = 10000
D = 128
N_PAD = 10240
E = 320000
E_PAD = 327680
NC = 2
NS = 16
CHUNK = 128
EPC = E_PAD // NC
EPS = EPC // NS
NCHUNKS = EPS // CHUNK
RPS = N_PAD // NS
DW = 16
BLK = 1024
GRID = N_PAD // BLK


def _sc_mesh():
    return plsc.VectorSubcoreMesh(core_axis_name="c", subcore_axis_name="s",
                                  num_cores=NC, num_subcores=NS)


N_DEG = 2 * N_PAD
DEG_RPS = N_DEG // NS
DEG_EPS = 2 * EPS
DEG_NCHUNKS = DEG_EPS // CHUNK


@functools.cache
def _get_deg_kernel():
    return functools.partial(
        pl.kernel,
        out_type=jax.ShapeDtypeStruct((NC * N_DEG,), jnp.float32),
        mesh=_sc_mesh(),
        scratch_types=[
            pltpu.VMEM_SHARED((N_DEG,), jnp.float32),
            pltpu.VMEM((CHUNK,), jnp.int32),
            pltpu.VMEM((CHUNK,), jnp.float32),
            pltpu.VMEM((DEG_RPS,), jnp.float32),
        ],
    )(_deg_body)


def _deg_body(sd_idx, ones_h, zeros_h, deg_out,
              deg_s, idx_v, ones_v, buf):
    c = lax.axis_index("c")
    s = lax.axis_index("s")
    r0 = s * DEG_RPS
    pltpu.sync_copy(ones_h, ones_v)
    pltpu.sync_copy(zeros_h, buf)
    pltpu.sync_copy(buf, deg_s.at[pl.ds(r0, DEG_RPS)])
    plsc.subcore_barrier()

    def body(t, carry):
        base = c * (2 * EPC) + s * DEG_EPS + t * CHUNK
        pltpu.sync_copy(sd_idx.at[pl.ds(base, CHUNK)], idx_v)
        pltpu.sync_copy(ones_v, deg_s.at[idx_v], add=True)
        return carry

    lax.fori_loop(0, DEG_NCHUNKS, body, 0)
    plsc.subcore_barrier()
    pltpu.sync_copy(deg_s.at[pl.ds(r0, DEG_RPS)], buf)
    pltpu.sync_copy(buf, deg_out.at[pl.ds(c * N_DEG + r0, DEG_RPS)])


NBUF = 2
IBUF = 4


@functools.cache
def _get_edge_kernel():
    return functools.partial(
        pl.kernel,
        out_type=jax.ShapeDtypeStruct((NC * N_PAD, D), jnp.float32),
        mesh=_sc_mesh(),
        scratch_types=[
            pltpu.VMEM_SHARED((N_PAD, D), jnp.float32),
            pltpu.VMEM((IBUF, CHUNK), jnp.int32),
            pltpu.VMEM((IBUF, CHUNK), jnp.int32),
            pltpu.VMEM((NBUF, CHUNK, D), jnp.float32),
            pltpu.SemaphoreType.DMA((IBUF,)),
            pltpu.SemaphoreType.DMA((NBUF,)),
            pltpu.SemaphoreType.DMA((NBUF,)),
        ],
    )(_edge_body)


def _edge_body(hpre, src2, dst2, zeros_h, acc_out,
               acc_s, isrc, idst, rows, si, sg, ss):
    c = lax.axis_index("c")
    s = lax.axis_index("s")
    r0 = s * RPS
    w0 = (c * NS + s) * NCHUNKS

    def i_start(t, bi):
        pltpu.async_copy(src2.at[w0 + t], isrc.at[bi], si.at[bi])
        pltpu.async_copy(dst2.at[w0 + t], idst.at[bi], si.at[bi])

    def i_wait(t, bi):
        pltpu.make_async_copy(src2.at[w0 + t], isrc.at[bi], si.at[bi]).wait()
        pltpu.make_async_copy(dst2.at[w0 + t], idst.at[bi], si.at[bi]).wait()

    pltpu.sync_copy(zeros_h, rows.at[0])
    for k in range(RPS // CHUNK):
        pltpu.sync_copy(rows.at[0], acc_s.at[pl.ds(r0 + k * CHUNK, CHUNK)])
    plsc.subcore_barrier()

    i_start(0, 0)
    i_start(1, 1)
    i_wait(0, 0)
    pltpu.async_copy(hpre.at[isrc.at[0]], rows.at[0], sg.at[0])

    def body(t, carry):
        b = lax.rem(t, NBUF)
        nb = lax.rem(t + 1, NBUF)

        @pl.when(t >= 1)
        def _():
            pltpu.make_async_copy(rows.at[nb], acc_s.at[idst.at[lax.rem(t - 1, IBUF)]],
                                  ss.at[nb]).wait()
        @pl.when(t + 2 < NCHUNKS)
        def _():
            i_start(t + 2, lax.rem(t + 2, IBUF))
        @pl.when(t + 1 < NCHUNKS)
        def _():
            i_wait(t + 1, lax.rem(t + 1, IBUF))
            pltpu.async_copy(hpre.at[isrc.at[lax.rem(t + 1, IBUF)]],
                             rows.at[nb], sg.at[nb])
        pltpu.make_async_copy(hpre.at[isrc.at[lax.rem(t, IBUF)]],
                              rows.at[b], sg.at[b]).wait()
        pltpu.async_copy(rows.at[b], acc_s.at[idst.at[lax.rem(t, IBUF)]],
                         ss.at[b], add=True)
        return carry

    lax.fori_loop(0, NCHUNKS, body, 0)
    lastb = (NCHUNKS - 1) % NBUF
    pltpu.make_async_copy(rows.at[lastb],
                          acc_s.at[idst.at[(NCHUNKS - 1) % IBUF]],
                          ss.at[lastb]).wait()
    plsc.subcore_barrier()
    for k in range(RPS // CHUNK):
        pltpu.sync_copy(acc_s.at[pl.ds(r0 + k * CHUNK, CHUNK)], rows.at[0])
        pltpu.sync_copy(rows.at[0], acc_out.at[pl.ds(c * N_PAD + r0 + k * CHUNK, CHUNK)])


def _prep_body(x_ref, dego, degi, hpre_ref, sin_ref, sout_ref):
    so = lax.rsqrt(jnp.maximum(dego[...], 1.0))
    si = lax.rsqrt(jnp.maximum(degi[...], 1.0))
    hpre_ref[...] = x_ref[...] * so
    sout_ref[...] = so
    sin_ref[...] = si


def _tc_prep(x_pad, deg_all):
    return pl.pallas_call(
        _prep_body,
        grid=(GRID,),
        in_specs=[
            pl.BlockSpec((BLK, D), lambda i: (i, 0)),
            pl.BlockSpec((BLK, 1), lambda i: (i, 0)),
            pl.BlockSpec((BLK, 1), lambda i: (i + 3 * GRID, 0)),
        ],
        out_specs=[
            pl.BlockSpec((BLK, D), lambda i: (i, 0)),
            pl.BlockSpec((BLK, 1), lambda i: (i, 0)),
            pl.BlockSpec((BLK, 1), lambda i: (i, 0)),
        ],
        out_shape=[
            jax.ShapeDtypeStruct((N_PAD, D), jnp.float32),
            jax.ShapeDtypeStruct((N_PAD, 1), jnp.float32),
            jax.ShapeDtypeStruct((N_PAD, 1), jnp.float32),
        ],
    )(x_pad, deg_all, deg_all)


def _layer_body(p0, p1, sin_ref, sout_ref, w_ref, b_ref, out_ref):
    agg = (p0[...] + p1[...]) * sin_ref[...]
    h = jnp.dot(agg, w_ref[...], preferred_element_type=jnp.float32) + b_ref[...]
    out_ref[...] = jnp.maximum(h, 0.0) * sout_ref[...]


def _tc_layer(acc_partials, sin, sout, w, b):
    return pl.pallas_call(
        _layer_body,
        grid=(GRID,),
        in_specs=[
            pl.BlockSpec((BLK, D), lambda i: (i, 0)),
            pl.BlockSpec((BLK, D), lambda i: (i + GRID, 0)),
            pl.BlockSpec((BLK, 1), lambda i: (i, 0)),
            pl.BlockSpec((BLK, 1), lambda i: (i, 0)),
            pl.BlockSpec((D, D), lambda i: (0, 0)),
            pl.BlockSpec((1, D), lambda i: (0, 0)),
        ],
        out_specs=pl.BlockSpec((BLK, D), lambda i: (i, 0)),
        out_shape=jax.ShapeDtypeStruct((N_PAD, D), jnp.float32),
    )(acc_partials, acc_partials, sin, sout, w, b.reshape(1, D))


def _head_body(p0, p1, sin_ref, w_ref, b_ref, wr_ref, br_ref, out_ref, acc_ref):
    i = pl.program_id(0)
    agg = (p0[...] + p1[...]) * sin_ref[...]
    h = jnp.dot(agg, w_ref[...], preferred_element_type=jnp.float32) + b_ref[...]
    h = jnp.maximum(h, 0.0)
    row = i * BLK + lax.broadcasted_iota(jnp.int32, (BLK, 1), 0)
    h = jnp.where(row < N_NODES, h, 0.0)
    colsum = jnp.sum(h, axis=0, keepdims=True)

    @pl.when(i == 0)
    def _():
        acc_ref[...] = colsum

    @pl.when(i > 0)
    def _():
        acc_ref[...] = acc_ref[...] + colsum

    @pl.when(i == GRID - 1)
    def _():
        hg = acc_ref[...] * (1.0 / N_NODES)
        out_ref[...] = (jnp.dot(hg, wr_ref[...],
                                preferred_element_type=jnp.float32)
                        + br_ref[...])


def _tc_head(acc_partials, sin, w, b, wr, br):
    return pl.pallas_call(
        _head_body,
        grid=(GRID,),
        in_specs=[
            pl.BlockSpec((BLK, D), lambda i: (i, 0)),
            pl.BlockSpec((BLK, D), lambda i: (i + GRID, 0)),
            pl.BlockSpec((BLK, 1), lambda i: (i, 0)),
            pl.BlockSpec((D, D), lambda i: (0, 0)),
            pl.BlockSpec((1, D), lambda i: (0, 0)),
            pl.BlockSpec((D, 1), lambda i: (0, 0)),
            pl.BlockSpec((1, 1), lambda i: (0, 0)),
        ],
        out_specs=pl.BlockSpec((1, 1), lambda i: (0, 0)),
        out_shape=jax.ShapeDtypeStruct((1, 1), jnp.float32),
        scratch_shapes=[pltpu.VMEM((1, D), jnp.float32)],
    )(acc_partials, acc_partials, sin, w, b.reshape(1, D), wr,
      br.reshape(1, 1))


def kernel(x, edge_index, W0, b0, W1, b1, W2, b2, Wr, br):
    src = edge_index[0].astype(jnp.int32)
    dst = edge_index[1].astype(jnp.int32)
    pad_idx = jnp.full((E_PAD - E,), N_NODES, dtype=jnp.int32)
    srcp = jnp.concatenate([src, pad_idx])
    dstp = jnp.concatenate([dst, pad_idx])
    x_pad = jnp.pad(x, ((0, N_PAD - N_NODES), (0, 0)))

    ones_h = jnp.ones((CHUNK,), jnp.float32)
    zeros_deg = jnp.zeros((DEG_RPS,), jnp.float32)
    zeros_row = jnp.zeros((CHUNK, D), jnp.float32)
    sd_idx = jnp.concatenate([srcp, dstp + N_PAD])

    deg_kernel = _get_deg_kernel()
    edge_kernel = _get_edge_kernel()
    deg_all = deg_kernel(sd_idx, ones_h, zeros_deg)
    hpre, sin, sout = _tc_prep(x_pad, deg_all.reshape(NC * N_DEG, 1))

    src2 = srcp.reshape(E_PAD // CHUNK, CHUNK)
    dst2 = dstp.reshape(E_PAD // CHUNK, CHUNK)
    acc1 = edge_kernel(hpre, src2, dst2, zeros_row)
    hpre2 = _tc_layer(acc1, sin, sout, W0, b0)
    acc2 = edge_kernel(hpre2, src2, dst2, zeros_row)
    hpre3 = _tc_layer(acc2, sin, sout, W1, b1)
    acc3 = edge_kernel(hpre3, src2, dst2, zeros_row)
    return _tc_head(acc3, sin, W2, b2, Wr, br)


# speedup vs baseline: 4.0035x; 1.2208x over previous
import functools

import jax
import jax.numpy as jnp
from jax import lax
from jax.experimental import pallas as pl
from jax.experimental.pallas import tpu as pltpu
from jax.experimental.pallas import tpu_sc as plsc

N_NODES = 10000
D = 128
N_PAD = 10240
E = 320000
E_PAD = 327680
NC = 2
NS = 16
CHUNK = 128
EPC = E_PAD // NC
EPS = EPC // NS
NCHUNKS = EPS // CHUNK
RPS = N_PAD // NS
DW = 16
BLK = 1024
GRID = N_PAD // BLK


def _sc_mesh():
    return plsc.VectorSubcoreMesh(core_axis_name="c", subcore_axis_name="s",
                                  num_cores=NC, num_subcores=NS)


N_DEG = 2 * N_PAD
DEG_RPS = N_DEG // NS
DEG_EPS = 2 * EPS
DEG_NCHUNKS = DEG_EPS // CHUNK


@functools.cache
def _get_deg_kernel():
    return functools.partial(
        pl.kernel,
        out_type=jax.ShapeDtypeStruct((NC * N_DEG,), jnp.float32),
        mesh=_sc_mesh(),
        scratch_types=[
            pltpu.VMEM_SHARED((N_DEG,), jnp.float32),
            pltpu.VMEM((CHUNK,), jnp.int32),
            pltpu.VMEM((CHUNK,), jnp.float32),
            pltpu.VMEM((DEG_RPS,), jnp.float32),
        ],
    )(_deg_body)


def _deg_body(sd_idx, ones_h, zeros_h, deg_out,
              deg_s, idx_v, ones_v, buf):
    c = lax.axis_index("c")
    s = lax.axis_index("s")
    r0 = s * DEG_RPS
    pltpu.sync_copy(ones_h, ones_v)
    pltpu.sync_copy(zeros_h, buf)
    pltpu.sync_copy(buf, deg_s.at[pl.ds(r0, DEG_RPS)])
    plsc.subcore_barrier()

    def body(t, carry):
        base = c * (2 * EPC) + s * DEG_EPS + t * CHUNK
        pltpu.sync_copy(sd_idx.at[pl.ds(base, CHUNK)], idx_v)
        pltpu.sync_copy(ones_v, deg_s.at[idx_v], add=True)
        return carry

    lax.fori_loop(0, DEG_NCHUNKS, body, 0)
    plsc.subcore_barrier()
    pltpu.sync_copy(deg_s.at[pl.ds(r0, DEG_RPS)], buf)
    pltpu.sync_copy(buf, deg_out.at[pl.ds(c * N_DEG + r0, DEG_RPS)])


NBUF = 2
IBUF = 4


@functools.cache
def _get_edge_kernel():
    return functools.partial(
        pl.kernel,
        out_type=jax.ShapeDtypeStruct((NC * N_PAD, D), jnp.float32),
        mesh=_sc_mesh(),
        scratch_types=[
            pltpu.VMEM_SHARED((N_PAD, D), jnp.float32),
            pltpu.VMEM((IBUF, CHUNK), jnp.int32),
            pltpu.VMEM((IBUF, CHUNK), jnp.int32),
            pltpu.VMEM((NBUF, CHUNK, D), jnp.float32),
            pltpu.SemaphoreType.DMA((IBUF,)),
            pltpu.SemaphoreType.DMA((NBUF,)),
            pltpu.SemaphoreType.DMA((NBUF,)),
        ],
    )(_edge_body)


def _edge_body(hpre, src2, dst2, zeros_h, acc_out,
               acc_s, isrc, idst, rows, si, sg, ss):
    c = lax.axis_index("c")
    s = lax.axis_index("s")
    r0 = s * RPS
    w0 = (c * NS + s) * NCHUNKS

    def i_start(t, bi):
        pltpu.async_copy(src2.at[w0 + t], isrc.at[bi], si.at[bi])
        pltpu.async_copy(dst2.at[w0 + t], idst.at[bi], si.at[bi])

    def i_wait(t, bi):
        pltpu.make_async_copy(src2.at[w0 + t], isrc.at[bi], si.at[bi]).wait()
        pltpu.make_async_copy(dst2.at[w0 + t], idst.at[bi], si.at[bi]).wait()

    pltpu.sync_copy(zeros_h, rows.at[0])
    for k in range(RPS // CHUNK):
        pltpu.sync_copy(rows.at[0], acc_s.at[pl.ds(r0 + k * CHUNK, CHUNK)])
    plsc.subcore_barrier()

    i_start(0, 0)
    i_start(1, 1)
    i_wait(0, 0)
    pltpu.async_copy(hpre.at[isrc.at[0]], rows.at[0], sg.at[0])

    def body(t, carry):
        b = lax.rem(t, NBUF)
        nb = lax.rem(t + 1, NBUF)

        @pl.when(t >= 1)
        def _():
            pltpu.make_async_copy(rows.at[nb], acc_s.at[idst.at[lax.rem(t - 1, IBUF)]],
                                  ss.at[nb]).wait()
        @pl.when(t + 2 < NCHUNKS)
        def _():
            i_start(t + 2, lax.rem(t + 2, IBUF))
        @pl.when(t + 1 < NCHUNKS)
        def _():
            i_wait(t + 1, lax.rem(t + 1, IBUF))
            pltpu.async_copy(hpre.at[isrc.at[lax.rem(t + 1, IBUF)]],
                             rows.at[nb], sg.at[nb])
        pltpu.make_async_copy(hpre.at[isrc.at[lax.rem(t, IBUF)]],
                              rows.at[b], sg.at[b]).wait()
        pltpu.async_copy(rows.at[b], acc_s.at[idst.at[lax.rem(t, IBUF)]],
                         ss.at[b], add=True)
        return carry

    lax.fori_loop(0, NCHUNKS, body, 0)
    lastb = (NCHUNKS - 1) % NBUF
    pltpu.make_async_copy(rows.at[lastb],
                          acc_s.at[idst.at[(NCHUNKS - 1) % IBUF]],
                          ss.at[lastb]).wait()
    plsc.subcore_barrier()
    for k in range(RPS // CHUNK):
        pltpu.sync_copy(acc_s.at[pl.ds(r0 + k * CHUNK, CHUNK)], rows.at[0])
        pltpu.sync_copy(rows.at[0], acc_out.at[pl.ds(c * N_PAD + r0 + k * CHUNK, CHUNK)])


def _prep_body(x_ref, dego, degi, hpre_ref, sin_ref, sout_ref):
    so = lax.rsqrt(jnp.maximum(dego[...], 1.0))
    si = lax.rsqrt(jnp.maximum(degi[...], 1.0))
    hpre_ref[...] = x_ref[...] * so
    sout_ref[...] = so
    sin_ref[...] = si


def _tc_prep(x_pad, deg_all):
    return pl.pallas_call(
        _prep_body,
        grid=(GRID,),
        in_specs=[
            pl.BlockSpec((BLK, D), lambda i: (i, 0)),
            pl.BlockSpec((BLK, 1), lambda i: (i, 0)),
            pl.BlockSpec((BLK, 1), lambda i: (i + 3 * GRID, 0)),
        ],
        out_specs=[
            pl.BlockSpec((BLK, D), lambda i: (i, 0)),
            pl.BlockSpec((BLK, 1), lambda i: (i, 0)),
            pl.BlockSpec((BLK, 1), lambda i: (i, 0)),
        ],
        out_shape=[
            jax.ShapeDtypeStruct((N_PAD, D), jnp.float32),
            jax.ShapeDtypeStruct((N_PAD, 1), jnp.float32),
            jax.ShapeDtypeStruct((N_PAD, 1), jnp.float32),
        ],
    )(x_pad, deg_all, deg_all)


def _layer_body(p0, p1, sin_ref, sout_ref, w_ref, b_ref, out_ref):
    agg = (p0[...] + p1[...]) * sin_ref[...]
    h = jnp.dot(agg, w_ref[...], preferred_element_type=jnp.float32) + b_ref[...]
    out_ref[...] = jnp.maximum(h, 0.0) * sout_ref[...]


def _tc_layer(acc_partials, sin, sout, w, b):
    return pl.pallas_call(
        _layer_body,
        grid=(GRID,),
        in_specs=[
            pl.BlockSpec((BLK, D), lambda i: (i, 0)),
            pl.BlockSpec((BLK, D), lambda i: (i + GRID, 0)),
            pl.BlockSpec((BLK, 1), lambda i: (i, 0)),
            pl.BlockSpec((BLK, 1), lambda i: (i, 0)),
            pl.BlockSpec((D, D), lambda i: (0, 0)),
            pl.BlockSpec((1, D), lambda i: (0, 0)),
        ],
        out_specs=pl.BlockSpec((BLK, D), lambda i: (i, 0)),
        out_shape=jax.ShapeDtypeStruct((N_PAD, D), jnp.float32),
    )(acc_partials, acc_partials, sin, sout, w, b.reshape(1, D))


def _head_body(p0, p1, sin_ref, w_ref, b_ref, wr_row_ref, br_ref, out_ref, acc_ref):
    i = pl.program_id(0)
    agg = (p0[...] + p1[...]) * sin_ref[...]
    h = jnp.dot(agg, w_ref[...], preferred_element_type=jnp.float32) + b_ref[...]
    h = jnp.maximum(h, 0.0)
    row = i * BLK + lax.broadcasted_iota(jnp.int32, (BLK, 1), 0)
    h = jnp.where(row < N_NODES, h, 0.0)
    colsum = jnp.sum(h, axis=0, keepdims=True)

    @pl.when(i == 0)
    def _():
        acc_ref[...] = colsum

    @pl.when(i > 0)
    def _():
        acc_ref[...] = acc_ref[...] + colsum

    @pl.when(i == GRID - 1)
    def _():
        hg = acc_ref[...] * (1.0 / N_NODES)
        out_ref[...] = (jnp.sum(hg * wr_row_ref[...], axis=1, keepdims=True)
                        + br_ref[...])


def _tc_head(acc_partials, sin, w, b, wr, br):
    return pl.pallas_call(
        _head_body,
        grid=(GRID,),
        in_specs=[
            pl.BlockSpec((BLK, D), lambda i: (i, 0)),
            pl.BlockSpec((BLK, D), lambda i: (i + GRID, 0)),
            pl.BlockSpec((BLK, 1), lambda i: (i, 0)),
            pl.BlockSpec((D, D), lambda i: (0, 0)),
            pl.BlockSpec((1, D), lambda i: (0, 0)),
            pl.BlockSpec((1, D), lambda i: (0, 0)),
            pl.BlockSpec((1, 1), lambda i: (0, 0)),
        ],
        out_specs=pl.BlockSpec((1, 1), lambda i: (0, 0)),
        out_shape=jax.ShapeDtypeStruct((1, 1), jnp.float32),
        scratch_shapes=[pltpu.VMEM((1, D), jnp.float32)],
    )(acc_partials, acc_partials, sin, w, b.reshape(1, D), wr.reshape(1, D),
      br.reshape(1, 1))


def kernel(x, edge_index, W0, b0, W1, b1, W2, b2, Wr, br):
    src = edge_index[0].astype(jnp.int32)
    dst = edge_index[1].astype(jnp.int32)
    pad_idx = jnp.full((E_PAD - E,), N_NODES, dtype=jnp.int32)
    srcp = jnp.concatenate([src, pad_idx])
    dstp = jnp.concatenate([dst, pad_idx])
    x_pad = jnp.pad(x, ((0, N_PAD - N_NODES), (0, 0)))

    ones_h = jnp.ones((CHUNK,), jnp.float32)
    zeros_deg = jnp.zeros((DEG_RPS,), jnp.float32)
    zeros_row = jnp.zeros((CHUNK, D), jnp.float32)
    sd_idx = jnp.concatenate([srcp, dstp + N_PAD])

    deg_kernel = _get_deg_kernel()
    edge_kernel = _get_edge_kernel()
    deg_all = deg_kernel(sd_idx, ones_h, zeros_deg)
    hpre, sin, sout = _tc_prep(x_pad, deg_all.reshape(NC * N_DEG, 1))

    src2 = srcp.reshape(E_PAD // CHUNK, CHUNK)
    dst2 = dstp.reshape(E_PAD // CHUNK, CHUNK)
    acc1 = edge_kernel(hpre, src2, dst2, zeros_row)
    hpre2 = _tc_layer(acc1, sin, sout, W0, b0)
    acc2 = edge_kernel(hpre2, src2, dst2, zeros_row)
    hpre3 = _tc_layer(acc2, sin, sout, W1, b1)
    acc3 = edge_kernel(hpre3, src2, dst2, zeros_row)
    return _tc_head(acc3, sin, W2, b2, Wr, br)
